# baseline (speedup 1.0000x reference)
import functools

import jax
import jax.numpy as jnp
from jax import lax
from jax.experimental import pallas as pl
from jax.experimental.pallas import tpu as pltpu
from jax.experimental.pallas import tpu_sc as plsc

BM = 512
WIN = 2736
NWIN = 3


def _round_bf16(x):
    return x.astype(jnp.bfloat16).astype(jnp.float32)


def _argmin_body(x_ref, e_ref, zsq_ref, esq_ref, out_ref, nvs, neg0, neg1):
    n = pl.program_id(1)

    x = x_ref[...]
    e = e_ref[...].reshape(WIN, x.shape[1])
    esq = esq_ref[...].reshape(1, WIN)
    zsq = zsq_ref[...]
    mm = lax.dot_general(x, e, (((1,), (1,)), ((), ())),
                         preferred_element_type=jnp.float32)
    neg = 2.0 * mm - (zsq + esq)
    nv = jnp.max(neg, axis=1, keepdims=True)

    @pl.when(n == 0)
    def _():
        neg0[...] = neg
        nvs[:, 0:1] = nv

    @pl.when(n == 1)
    def _():
        neg1[...] = neg
        nvs[:, 1:2] = nv

    @pl.when(n == NWIN - 1)
    def _():
        nv0 = nvs[:, 0:1]
        nv1 = nvs[:, 1:2]
        nv2 = nv
        w = jnp.zeros_like(nv0, dtype=jnp.int32)
        acc = _round_bf16(nv0)
        keep1 = acc >= nv1
        w = jnp.where(keep1, w, 1)
        acc = _round_bf16(jnp.where(keep1, acc, nv1))
        keep2 = acc >= nv2
        w = jnp.where(keep2, w, 2)
        nv_sel = jnp.where(w == 0, nv0, jnp.where(w == 1, nv1, nv2))
        neg_sel = jnp.where(w == 0, neg0[...],
                            jnp.where(w == 1, neg1[...], neg))
        lanes = lax.broadcasted_iota(jnp.int32, neg.shape, 1)
        loc = jnp.min(jnp.where(neg_sel == nv_sel, lanes, jnp.int32(2**30)),
                      axis=1, keepdims=True)
        out_ref[...] = loc + w * WIN


def _argmin_call(flat, ew_w, zsq, esq_w):
    m, k = flat.shape
    grid = (m // BM, NWIN)
    return pl.pallas_call(
        _argmin_body,
        grid=grid,
        in_specs=[
            pl.BlockSpec((BM, k), lambda mi, ni: (mi, 0)),
            pl.BlockSpec((1, WIN, k), lambda mi, ni: (ni, 0, 0)),
            pl.BlockSpec((BM, 1), lambda mi, ni: (mi, 0)),
            pl.BlockSpec((1, 1, WIN), lambda mi, ni: (ni, 0, 0)),
        ],
        out_specs=pl.BlockSpec((BM, 1), lambda mi, ni: (mi, 0)),
        out_shape=jax.ShapeDtypeStruct((m, 1), jnp.int32),
        scratch_shapes=[
            pltpu.VMEM((BM, 8), jnp.float32),
            pltpu.VMEM((BM, WIN), jnp.float32),
            pltpu.VMEM((BM, WIN), jnp.float32),
        ],
        compiler_params=pltpu.CompilerParams(
            dimension_semantics=("parallel", "arbitrary")),
    )(flat, ew_w, zsq, esq_w)


def _gather_call(ew, idx_flat):
    info = plsc.get_sparse_core_info()
    nw = info.num_cores * info.num_subcores
    b = idx_flat.shape[0]
    d = ew.shape[1]
    b_per_w = b // nw
    ch = 128
    nch = b_per_w // ch
    mesh = plsc.VectorSubcoreMesh(core_axis_name="c", subcore_axis_name="s")

    @functools.partial(
        pl.kernel,
        mesh=mesh,
        out_type=jax.ShapeDtypeStruct((b, d), jnp.float32),
        scratch_types=[
            pltpu.VMEM((ch,), jnp.int32),
            pltpu.VMEM((ch, d), jnp.float32),
            pltpu.SemaphoreType.DMA,
        ],
    )
    def k(ew_hbm, idx_hbm, out_hbm, idx_v, rows_v, sem):
        wid = lax.axis_index("s") * info.num_cores + lax.axis_index("c")
        base = wid * b_per_w
        for c in range(nch):
            off = base + c * ch
            pltpu.sync_copy(idx_hbm.at[pl.ds(off, ch)], idx_v)
            pltpu.async_copy(ew_hbm.at[idx_v], rows_v, sem).wait()
            pltpu.sync_copy(rows_v, out_hbm.at[pl.ds(off, ch)])

    return k(ew, idx_flat)


def kernel(inputs, embed_weight):
    n_embed, dim = embed_weight.shape
    flat = inputs.reshape(-1, dim)
    pad = NWIN * WIN - n_embed
    zsq = jnp.sum(flat ** 2, axis=1, keepdims=True)
    esq = jnp.sum(embed_weight ** 2, axis=1)
    esq_w = jnp.pad(esq, (0, pad), constant_values=jnp.inf).reshape(
        NWIN, 1, WIN)
    ew_w = jnp.pad(embed_weight, ((0, pad), (0, 0))).reshape(
        NWIN, WIN, dim)
    idx = _argmin_call(flat, ew_w, zsq, esq_w).reshape(-1)
    quantize = _gather_call(embed_weight, idx).reshape(inputs.shape)
    return quantize, idx.reshape(inputs.shape[:-1])


# speedup vs baseline: 1.0894x; 1.0894x over previous
import functools

import jax
import jax.numpy as jnp
from jax import lax
from jax.experimental import pallas as pl
from jax.experimental.pallas import tpu as pltpu
from jax.experimental.pallas import tpu_sc as plsc

BM = 1024
WIN = 2736
NWIN = 3


def _round_bf16(x):
    return x.astype(jnp.bfloat16).astype(jnp.float32)


def _argmin_body(x_ref, e_ref, zsq_ref, esq_ref, out_ref, acc_v, acc_i):
    n = pl.program_id(1)

    @pl.when(n == 0)
    def _():
        acc_v[...] = jnp.full_like(acc_v[...], -jnp.inf)
        acc_i[...] = jnp.zeros_like(acc_i[...])

    x = x_ref[...]
    e = e_ref[...].reshape(WIN, x.shape[1])
    esq = esq_ref[...].reshape(1, WIN)
    zsq = zsq_ref[...]
    mm = lax.dot_general(x, e, (((1,), (1,)), ((), ())),
                         preferred_element_type=jnp.float32)
    neg = 2.0 * mm - (zsq + esq)
    nv = jnp.max(neg, axis=1, keepdims=True)
    lanes = lax.broadcasted_iota(jnp.int32, neg.shape, 1)
    ni = jnp.min(jnp.where(neg == nv, lanes, jnp.int32(2**30)),
                 axis=1, keepdims=True) + n * WIN
    keep = acc_v[...] >= nv
    acc_i[...] = jnp.where(keep, acc_i[...], ni)
    acc_v[...] = _round_bf16(jnp.where(keep, acc_v[...], nv))

    @pl.when(n == NWIN - 1)
    def _():
        out_ref[...] = acc_i[...]


def _argmin_call(flat, ew_w, zsq, esq_w):
    m, k = flat.shape
    grid = (m // BM, NWIN)
    return pl.pallas_call(
        _argmin_body,
        grid=grid,
        in_specs=[
            pl.BlockSpec((BM, k), lambda mi, ni: (mi, 0)),
            pl.BlockSpec((1, WIN, k), lambda mi, ni: (ni, 0, 0)),
            pl.BlockSpec((BM, 1), lambda mi, ni: (mi, 0)),
            pl.BlockSpec((1, 1, WIN), lambda mi, ni: (ni, 0, 0)),
        ],
        out_specs=pl.BlockSpec((BM, 1), lambda mi, ni: (mi, 0)),
        out_shape=jax.ShapeDtypeStruct((m, 1), jnp.int32),
        scratch_shapes=[
            pltpu.VMEM((BM, 1), jnp.float32),
            pltpu.VMEM((BM, 1), jnp.int32),
        ],
        compiler_params=pltpu.CompilerParams(
            dimension_semantics=("parallel", "arbitrary")),
    )(flat, ew_w, zsq, esq_w)


def _gather_call(ew, idx_flat):
    info = plsc.get_sparse_core_info()
    nw = info.num_cores * info.num_subcores
    b = idx_flat.shape[0]
    d = ew.shape[1]
    b_per_w = b // nw
    ch = 128
    nch = b_per_w // ch
    mesh = plsc.VectorSubcoreMesh(core_axis_name="c", subcore_axis_name="s")

    @functools.partial(
        pl.kernel,
        mesh=mesh,
        out_type=jax.ShapeDtypeStruct((b, d), jnp.float32),
        scratch_types=[
            pltpu.VMEM((ch,), jnp.int32),
            pltpu.VMEM((ch, d), jnp.float32),
            pltpu.SemaphoreType.DMA,
        ],
    )
    def k(ew_hbm, idx_hbm, out_hbm, idx_v, rows_v, sem):
        wid = lax.axis_index("s") * info.num_cores + lax.axis_index("c")
        base = wid * b_per_w
        for c in range(nch):
            off = base + c * ch
            pltpu.sync_copy(idx_hbm.at[pl.ds(off, ch)], idx_v)
            pltpu.async_copy(ew_hbm.at[idx_v], rows_v, sem).wait()
            pltpu.sync_copy(rows_v, out_hbm.at[pl.ds(off, ch)])

    return k(ew, idx_flat)


def kernel(inputs, embed_weight):
    n_embed, dim = embed_weight.shape
    flat = inputs.reshape(-1, dim)
    pad = NWIN * WIN - n_embed
    zsq = jnp.sum(flat ** 2, axis=1, keepdims=True)
    esq = jnp.sum(embed_weight ** 2, axis=1)
    esq_w = jnp.pad(esq, (0, pad), constant_values=jnp.inf).reshape(
        NWIN, 1, WIN)
    ew_w = jnp.pad(embed_weight, ((0, pad), (0, 0))).reshape(
        NWIN, WIN, dim)
    idx = _argmin_call(flat, ew_w, zsq, esq_w).reshape(-1)
    quantize = _gather_call(embed_weight, idx).reshape(inputs.shape)
    return quantize, idx.reshape(inputs.shape[:-1])


# speedup vs baseline: 1.1273x; 1.0348x over previous
import functools

import jax
import jax.numpy as jnp
from jax import lax
from jax.experimental import pallas as pl
from jax.experimental.pallas import tpu as pltpu
from jax.experimental.pallas import tpu_sc as plsc

BM = 2048
WIN = 2736
NWIN = 3


def _round_bf16(x):
    return x.astype(jnp.bfloat16).astype(jnp.float32)


def _argmin_body(x_ref, e_ref, zsq_ref, esq_ref, out_ref, acc_v, acc_i):
    n = pl.program_id(1)

    @pl.when(n == 0)
    def _():
        acc_v[...] = jnp.full_like(acc_v[...], -jnp.inf)
        acc_i[...] = jnp.zeros_like(acc_i[...])

    x = x_ref[...]
    e = e_ref[...].reshape(WIN, x.shape[1])
    esq = esq_ref[...].reshape(1, WIN)
    zsq = zsq_ref[...]
    mm = lax.dot_general(x, e, (((1,), (1,)), ((), ())),
                         preferred_element_type=jnp.float32)
    neg = 2.0 * mm - (zsq + esq)
    nv = jnp.max(neg, axis=1, keepdims=True)
    lanes = lax.broadcasted_iota(jnp.int32, neg.shape, 1)
    ni = jnp.min(jnp.where(neg == nv, lanes, jnp.int32(2**30)),
                 axis=1, keepdims=True) + n * WIN
    keep = acc_v[...] >= nv
    acc_i[...] = jnp.where(keep, acc_i[...], ni)
    acc_v[...] = _round_bf16(jnp.where(keep, acc_v[...], nv))

    @pl.when(n == NWIN - 1)
    def _():
        out_ref[...] = acc_i[...]


def _argmin_call(flat, ew_w, zsq, esq_w):
    m, k = flat.shape
    grid = (m // BM, NWIN)
    return pl.pallas_call(
        _argmin_body,
        grid=grid,
        in_specs=[
            pl.BlockSpec((BM, k), lambda mi, ni: (mi, 0)),
            pl.BlockSpec((1, WIN, k), lambda mi, ni: (ni, 0, 0)),
            pl.BlockSpec((BM, 1), lambda mi, ni: (mi, 0)),
            pl.BlockSpec((1, 1, WIN), lambda mi, ni: (ni, 0, 0)),
        ],
        out_specs=pl.BlockSpec((BM, 1), lambda mi, ni: (mi, 0)),
        out_shape=jax.ShapeDtypeStruct((m, 1), jnp.int32),
        scratch_shapes=[
            pltpu.VMEM((BM, 1), jnp.float32),
            pltpu.VMEM((BM, 1), jnp.int32),
        ],
        compiler_params=pltpu.CompilerParams(
            dimension_semantics=("parallel", "arbitrary")),
    )(flat, ew_w, zsq, esq_w)


def _gather_call(ew, idx_flat):
    info = plsc.get_sparse_core_info()
    nw = info.num_cores * info.num_subcores
    b = idx_flat.shape[0]
    d = ew.shape[1]
    b_per_w = b // nw
    ch = 128
    nch = b_per_w // ch
    mesh = plsc.VectorSubcoreMesh(core_axis_name="c", subcore_axis_name="s")

    @functools.partial(
        pl.kernel,
        mesh=mesh,
        out_type=jax.ShapeDtypeStruct((b, d), jnp.float32),
        scratch_types=[
            pltpu.VMEM((ch,), jnp.int32),
            pltpu.VMEM((ch, d), jnp.float32),
            pltpu.SemaphoreType.DMA,
        ],
    )
    def k(ew_hbm, idx_hbm, out_hbm, idx_v, rows_v, sem):
        wid = lax.axis_index("s") * info.num_cores + lax.axis_index("c")
        base = wid * b_per_w
        for c in range(nch):
            off = base + c * ch
            pltpu.sync_copy(idx_hbm.at[pl.ds(off, ch)], idx_v)
            pltpu.async_copy(ew_hbm.at[idx_v], rows_v, sem).wait()
            pltpu.sync_copy(rows_v, out_hbm.at[pl.ds(off, ch)])

    return k(ew, idx_flat)


def kernel(inputs, embed_weight):
    n_embed, dim = embed_weight.shape
    flat = inputs.reshape(-1, dim)
    pad = NWIN * WIN - n_embed
    zsq = jnp.sum(flat ** 2, axis=1, keepdims=True)
    esq = jnp.sum(embed_weight ** 2, axis=1)
    esq_w = jnp.pad(esq, (0, pad), constant_values=jnp.inf).reshape(
        NWIN, 1, WIN)
    ew_w = jnp.pad(embed_weight, ((0, pad), (0, 0))).reshape(
        NWIN, WIN, dim)
    idx = _argmin_call(flat, ew_w, zsq, esq_w).reshape(-1)
    quantize = _gather_call(embed_weight, idx).reshape(inputs.shape)
    return quantize, idx.reshape(inputs.shape[:-1])
